# baseline (speedup 1.0000x reference)
import jax
import jax.numpy as jnp
from jax import lax
from jax.experimental import pallas as pl
from jax.experimental.pallas import tpu as pltpu
from jax.experimental.pallas import tpu_sc as plsc

BN_EPS = 1e-5
NC = 2
NS = 16
CHUNK = 128
K = 4


def _make_sc_agg(n, dh, cpw, acc_n, rows_per_tile, num_groups):
    mesh = plsc.VectorSubcoreMesh(
        core_axis_name="c", subcore_axis_name="s", num_cores=NC, num_subcores=NS
    )

    def body(h2_hbm, idxp_hbm, zeros_hbm, out_hbm,
             acc, idx_v, rows_v, gsem, ssem, isem, zsem):
        cid = lax.axis_index("c")
        sid = lax.axis_index("s")
        row0 = sid * rows_per_tile
        zcopy = pltpu.make_async_copy(
            zeros_hbm, acc.at[pl.ds(row0, rows_per_tile)], zsem)
        zcopy.start()

        base_c = sid * cpw

        def fire_gathers(b):
            for k in range(K):
                pltpu.async_copy(
                    h2_hbm.at[idx_v.at[b, k, 0]],
                    rows_v.at[b, pl.ds(k * CHUNK, CHUNK)], gsem)

        def drain_gathers(b):
            for k in range(K):
                pltpu.make_async_copy(
                    h2_hbm.at[idx_v.at[b, k, 0]],
                    rows_v.at[b, pl.ds(k * CHUNK, CHUNK)], gsem).wait()

        def fire_scatters(b):
            for k in range(K):
                pltpu.async_copy(
                    rows_v.at[b, pl.ds(k * CHUNK, CHUNK)],
                    acc.at[idx_v.at[b, k, 1]], ssem, add=True)

        def drain_scatters(b):
            for k in range(K):
                pltpu.make_async_copy(
                    rows_v.at[b, pl.ds(k * CHUNK, CHUNK)],
                    acc.at[idx_v.at[b, k, 1]], ssem).wait()

        pltpu.sync_copy(idxp_hbm.at[cid, pl.ds(base_c, K)], idx_v.at[0])
        fire_gathers(0)
        zcopy.wait()
        plsc.subcore_barrier()

        def step(g, carry):
            b = lax.rem(g, 2)
            nb = 1 - b

            @pl.when(g > 0)
            def _():
                drain_scatters(nb)

            @pl.when(g + 1 < num_groups)
            def _():
                pltpu.async_copy(
                    idxp_hbm.at[cid, pl.ds(base_c + (g + 1) * K, K)],
                    idx_v.at[nb], isem)

            drain_gathers(b)
            fire_scatters(b)

            @pl.when(g + 1 < num_groups)
            def _():
                pltpu.make_async_copy(
                    idxp_hbm.at[cid, pl.ds(base_c + (g + 1) * K, K)],
                    idx_v.at[nb], isem).wait()
                fire_gathers(nb)

            return carry

        lax.fori_loop(0, num_groups, step, 0)
        drain_scatters((num_groups - 1) % 2)
        plsc.subcore_barrier()
        pltpu.sync_copy(acc.at[pl.ds(row0, rows_per_tile)],
                        out_hbm.at[cid, pl.ds(row0, rows_per_tile)])

    kern = pl.kernel(
        body,
        out_type=jax.ShapeDtypeStruct((NC, acc_n, dh), jnp.float32),
        mesh=mesh,
        scratch_types=[
            pltpu.VMEM_SHARED((acc_n, dh), jnp.float32),
            pltpu.VMEM((2, K, 2, CHUNK), jnp.int32),
            pltpu.VMEM((2, K * CHUNK, dh), jnp.float32),
            pltpu.SemaphoreType.DMA,
            pltpu.SemaphoreType.DMA,
            pltpu.SemaphoreType.DMA,
            pltpu.SemaphoreType.DMA,
        ],
        compiler_params=pltpu.CompilerParams(use_tc_tiling_on_sc=False),
    )
    return kern


def _make_dense(n, d, dh, split_out, relu_out):
    def body(h2_ref, p_ref, w1_ref, b1_ref, w2_ref, b2_ref, g_ref, be_ref, o_ref):
        h = jnp.concatenate([h2_ref[:n, :], h2_ref[n:, :]], axis=1)
        agg = jnp.concatenate([p_ref[0, :n, :], p_ref[1, :n, :]], axis=1)
        z = h + agg
        z = jnp.dot(z, w1_ref[...], preferred_element_type=jnp.float32) + b1_ref[...]
        z = jnp.maximum(z, 0.0)
        z = jnp.dot(z, w2_ref[...], preferred_element_type=jnp.float32) + b2_ref[...]
        m = jnp.mean(z, axis=0, keepdims=True)
        dlt = z - m
        v = jnp.mean(dlt * dlt, axis=0, keepdims=True)
        zn = dlt * lax.rsqrt(v + BN_EPS) * g_ref[...] + be_ref[...]
        if relu_out:
            zn = jnp.maximum(zn, 0.0)
        if split_out:
            o_ref[:n, :] = zn[:, :dh]
            o_ref[n:, :] = zn[:, dh:]
        else:
            o_ref[...] = zn

    out_shape = (
        jax.ShapeDtypeStruct((2 * n, dh), jnp.float32)
        if split_out else jax.ShapeDtypeStruct((n, d), jnp.float32)
    )
    return pl.pallas_call(body, out_shape=out_shape)


def kernel(x, edge_index, W1, b1, W2, b2, gamma, beta):
    n, d = x.shape
    e = edge_index.shape[1]
    num_layers = W1.shape[0]
    dh = d // 2

    num_groups = -(-e // (CHUNK * NS * K))
    cpw = num_groups * K
    e_pad = cpw * CHUNK * NS
    nchunk = e_pad // CHUNK
    rows_per_tile = -(-(n + 1) // NS)
    rows_per_tile = -(-rows_per_tile // 8) * 8
    acc_n = rows_per_tile * NS

    src = edge_index[0]
    dst = edge_index[1]
    pad = e_pad - e
    srcp = jnp.concatenate([src, jnp.zeros((pad,), jnp.int32)]).reshape(nchunk, CHUNK)
    dstp = jnp.concatenate([dst, jnp.full((pad,), n, jnp.int32)]).reshape(nchunk, CHUNK)
    idx0 = jnp.stack([srcp, dstp], axis=1)
    idx1 = jnp.stack([srcp + n, dstp], axis=1)
    idxp = jnp.stack([idx0, idx1], axis=0)
    zeros = jnp.zeros((rows_per_tile, dh), jnp.float32)

    sc_agg = _make_sc_agg(n, dh, cpw, acc_n, rows_per_tile, num_groups)

    b1r = b1.reshape(num_layers, 1, d)
    b2r = b2.reshape(num_layers, 1, d)
    gr = gamma.reshape(num_layers, 1, d)
    br = beta.reshape(num_layers, 1, d)

    h2 = jnp.concatenate([x[:, :dh], x[:, dh:]], axis=0)
    for i in range(num_layers):
        parts = sc_agg(h2, idxp, zeros)
        last = i == num_layers - 1
        dense = _make_dense(n, d, dh, split_out=not last, relu_out=not last)
        h2 = dense(h2, parts, W1[i], b1r[i], W2[i], b2r[i], gr[i], br[i])
    return h2


# speedup vs baseline: 1.0933x; 1.0933x over previous
import jax
import jax.numpy as jnp
from jax import lax
from jax.experimental import pallas as pl
from jax.experimental.pallas import tpu as pltpu
from jax.experimental.pallas import tpu_sc as plsc

BN_EPS = 1e-5
NC = 2
NS = 16
CHUNK = 128
NSLOT = 3


def _make_sc_agg(n, d, cpw, acc_n, rows_per_tile, num_groups):
    mesh = plsc.VectorSubcoreMesh(
        core_axis_name="c", subcore_axis_name="s", num_cores=NC, num_subcores=NS
    )

    def body(h_hbm, idxp_hbm, zeros_hbm, out_hbm,
             acc, idx_v, rows_v, gsem, ssem, isem, zsem):
        cid = lax.axis_index("c")
        sid = lax.axis_index("s")
        row0 = sid * rows_per_tile
        zcopy = pltpu.make_async_copy(
            zeros_hbm, acc.at[pl.ds(row0, rows_per_tile)], zsem)
        zcopy.start()

        wid = sid * NC + cid
        base_c = wid * cpw

        def gather_desc(s):
            return pltpu.make_async_copy(
                h_hbm.at[idx_v.at[s, 0]], rows_v.at[s], gsem)

        def scatter_desc(s):
            return pltpu.make_async_copy(
                rows_v.at[s], acc.at[idx_v.at[s, 1]], ssem)

        def idx_desc(g, s):
            return pltpu.make_async_copy(
                idxp_hbm.at[base_c + g], idx_v.at[s], isem)

        pltpu.sync_copy(idxp_hbm.at[base_c], idx_v.at[0])
        gather_desc(0).start()
        zcopy.wait()
        plsc.subcore_barrier()

        def step(g, carry):
            s = lax.rem(g, NSLOT)
            s1 = lax.rem(g + 1, NSLOT)

            @pl.when(g >= 2)
            def _():
                scatter_desc(s1).wait()

            @pl.when(g + 1 < num_groups)
            def _():
                idx_desc(g + 1, s1).start()

            gather_desc(s).wait()
            scatter_desc(s).start(add=True)

            @pl.when(g + 1 < num_groups)
            def _():
                idx_desc(g + 1, s1).wait()
                gather_desc(s1).start()

            return carry

        lax.fori_loop(0, num_groups, step, 0)
        scatter_desc((num_groups - 2) % NSLOT).wait()
        scatter_desc((num_groups - 1) % NSLOT).wait()
        plsc.subcore_barrier()
        pltpu.sync_copy(acc.at[pl.ds(row0, rows_per_tile)],
                        out_hbm.at[cid, pl.ds(row0, rows_per_tile)])

    kern = pl.kernel(
        body,
        out_type=jax.ShapeDtypeStruct((NC, acc_n, d), jnp.float32),
        mesh=mesh,
        scratch_types=[
            pltpu.VMEM_SHARED((acc_n, d), jnp.float32),
            pltpu.VMEM((NSLOT, 2, CHUNK), jnp.int32),
            pltpu.VMEM((NSLOT, CHUNK, d), jnp.float32),
            pltpu.SemaphoreType.DMA,
            pltpu.SemaphoreType.DMA,
            pltpu.SemaphoreType.DMA,
            pltpu.SemaphoreType.DMA,
        ],
    )
    return kern


def _make_dense(n, d, relu_out):
    def body(h_ref, p_ref, w1_ref, b1_ref, w2_ref, b2_ref, g_ref, be_ref, o_ref):
        z = h_ref[...] + p_ref[0, :n, :] + p_ref[1, :n, :]
        z = jnp.dot(z, w1_ref[...], preferred_element_type=jnp.float32) + b1_ref[...]
        z = jnp.maximum(z, 0.0)
        z = jnp.dot(z, w2_ref[...], preferred_element_type=jnp.float32) + b2_ref[...]
        m = jnp.mean(z, axis=0, keepdims=True)
        dlt = z - m
        v = jnp.mean(dlt * dlt, axis=0, keepdims=True)
        zn = dlt * lax.rsqrt(v + BN_EPS) * g_ref[...] + be_ref[...]
        if relu_out:
            zn = jnp.maximum(zn, 0.0)
        o_ref[...] = zn

    return pl.pallas_call(
        body,
        out_shape=jax.ShapeDtypeStruct((n, d), jnp.float32),
    )


def kernel(x, edge_index, W1, b1, W2, b2, gamma, beta):
    n, d = x.shape
    e = edge_index.shape[1]
    num_layers = W1.shape[0]

    nw = NC * NS
    num_groups = -(-e // (CHUNK * nw))
    cpw = num_groups
    e_pad = cpw * CHUNK * nw
    nchunk = e_pad // CHUNK
    rows_per_tile = -(-(n + 1) // NS)
    rows_per_tile = -(-rows_per_tile // 8) * 8
    acc_n = rows_per_tile * NS

    src = edge_index[0]
    dst = edge_index[1]
    pad = e_pad - e
    srcp = jnp.concatenate([src, jnp.zeros((pad,), jnp.int32)]).reshape(nchunk, CHUNK)
    dstp = jnp.concatenate([dst, jnp.full((pad,), n, jnp.int32)]).reshape(nchunk, CHUNK)
    idxp = jnp.stack([srcp, dstp], axis=1)
    zeros = jnp.zeros((rows_per_tile, d), jnp.float32)

    sc_agg = _make_sc_agg(n, d, cpw, acc_n, rows_per_tile, num_groups)

    b1r = b1.reshape(num_layers, 1, d)
    b2r = b2.reshape(num_layers, 1, d)
    gr = gamma.reshape(num_layers, 1, d)
    br = beta.reshape(num_layers, 1, d)

    h = x
    for i in range(num_layers):
        parts = sc_agg(h, idxp, zeros)
        dense = _make_dense(n, d, relu_out=(i < num_layers - 1))
        h = dense(h, parts, W1[i], b1r[i], W2[i], b2r[i], gr[i], br[i])
    return h


# speedup vs baseline: 1.0955x; 1.0020x over previous
import jax
import jax.numpy as jnp
from jax import lax
from jax.experimental import pallas as pl
from jax.experimental.pallas import tpu as pltpu
from jax.experimental.pallas import tpu_sc as plsc

BN_EPS = 1e-5
NC = 2
NS = 16
CHUNK = 128
NSLOT = 3


def _make_sc_agg(n, d, cpw, acc_n, rows_per_tile, num_groups):
    mesh = plsc.VectorSubcoreMesh(
        core_axis_name="c", subcore_axis_name="s", num_cores=NC, num_subcores=NS
    )

    def body(h_hbm, idxp_hbm, zeros_hbm, out_hbm,
             acc, idx_v, rows_v, gsem, ssem, isem, zsem):
        cid = lax.axis_index("c")
        sid = lax.axis_index("s")
        row0 = sid * rows_per_tile
        zcopy = pltpu.make_async_copy(
            zeros_hbm, acc.at[pl.ds(row0, rows_per_tile)], zsem)
        zcopy.start()

        wid = sid * NC + cid
        base_c = wid * cpw

        def gather_desc(s):
            return pltpu.make_async_copy(
                h_hbm.at[idx_v.at[s, 0]], rows_v.at[s], gsem)

        def scatter_desc(s):
            return pltpu.make_async_copy(
                rows_v.at[s], acc.at[idx_v.at[s, 1]], ssem)

        def idx_desc(g, s):
            return pltpu.make_async_copy(
                idxp_hbm.at[base_c + g], idx_v.at[s], isem)

        pltpu.sync_copy(idxp_hbm.at[base_c], idx_v.at[0])
        gather_desc(0).start()
        zcopy.wait()
        plsc.subcore_barrier()

        def step(g, carry):
            s = lax.rem(g, NSLOT)
            s1 = lax.rem(g + 1, NSLOT)

            @pl.when(g >= 2)
            def _():
                scatter_desc(s1).wait()

            @pl.when(g + 1 < num_groups)
            def _():
                idx_desc(g + 1, s1).start()

            gather_desc(s).wait()
            scatter_desc(s).start(add=True)

            @pl.when(g + 1 < num_groups)
            def _():
                idx_desc(g + 1, s1).wait()
                gather_desc(s1).start()

            return carry

        lax.fori_loop(0, num_groups, step, 0)
        scatter_desc((num_groups - 2) % NSLOT).wait()
        scatter_desc((num_groups - 1) % NSLOT).wait()
        plsc.subcore_barrier()
        pltpu.sync_copy(acc.at[pl.ds(row0, rows_per_tile)],
                        out_hbm.at[cid, pl.ds(row0, rows_per_tile)])

    kern = pl.kernel(
        body,
        out_type=jax.ShapeDtypeStruct((NC, acc_n, d), jnp.float32),
        mesh=mesh,
        scratch_types=[
            pltpu.VMEM_SHARED((acc_n, d), jnp.float32),
            pltpu.VMEM((NSLOT, 2, CHUNK), jnp.int32),
            pltpu.VMEM((NSLOT, CHUNK, d), jnp.float32),
            pltpu.SemaphoreType.DMA,
            pltpu.SemaphoreType.DMA,
            pltpu.SemaphoreType.DMA,
            pltpu.SemaphoreType.DMA,
        ],
    )
    return kern


def _make_dense(n, d, relu_out):
    def body(h_ref, p_ref, w1_ref, b1_ref, w2_ref, b2_ref, g_ref, be_ref, o_ref):
        z = h_ref[...] + p_ref[0, :n, :] + p_ref[1, :n, :]
        z = jnp.dot(z, w1_ref[...], preferred_element_type=jnp.float32) + b1_ref[...]
        z = jnp.maximum(z, 0.0)
        z = jnp.dot(z, w2_ref[...], preferred_element_type=jnp.float32) + b2_ref[...]
        m = jnp.mean(z, axis=0, keepdims=True)
        dlt = z - m
        v = jnp.mean(dlt * dlt, axis=0, keepdims=True)
        zn = dlt / jnp.sqrt(v + BN_EPS) * g_ref[...] + be_ref[...]
        if relu_out:
            zn = jnp.maximum(zn, 0.0)
        o_ref[...] = zn

    return pl.pallas_call(
        body,
        out_shape=jax.ShapeDtypeStruct((n, d), jnp.float32),
    )


def kernel(x, edge_index, W1, b1, W2, b2, gamma, beta):
    n, d = x.shape
    e = edge_index.shape[1]
    num_layers = W1.shape[0]

    nw = NC * NS
    num_groups = -(-e // (CHUNK * nw))
    cpw = num_groups
    e_pad = cpw * CHUNK * nw
    nchunk = e_pad // CHUNK
    rows_per_tile = -(-(n + 1) // NS)
    rows_per_tile = -(-rows_per_tile // 8) * 8
    acc_n = rows_per_tile * NS

    src = edge_index[0]
    dst = edge_index[1]
    pad = e_pad - e
    srcp = jnp.concatenate([src, jnp.zeros((pad,), jnp.int32)]).reshape(nchunk, CHUNK)
    dstp = jnp.concatenate([dst, jnp.full((pad,), n, jnp.int32)]).reshape(nchunk, CHUNK)
    idxp = jnp.stack([srcp, dstp], axis=1)
    zeros = jnp.zeros((rows_per_tile, d), jnp.float32)

    sc_agg = _make_sc_agg(n, d, cpw, acc_n, rows_per_tile, num_groups)

    b1r = b1.reshape(num_layers, 1, d)
    b2r = b2.reshape(num_layers, 1, d)
    gr = gamma.reshape(num_layers, 1, d)
    br = beta.reshape(num_layers, 1, d)

    h = x
    for i in range(num_layers):
        parts = sc_agg(h, idxp, zeros)
        dense = _make_dense(n, d, relu_out=(i < num_layers - 1))
        h = dense(h, parts, W1[i], b1r[i], W2[i], b2r[i], gr[i], br[i])
    return h
